# baseline (speedup 1.0000x reference)
import jax
import jax.numpy as jnp
from jax import lax
from jax.experimental import pallas as pl
from jax.experimental.pallas import tpu as pltpu
from jax.experimental.pallas import tpu_sc as plsc

_NC, _NS = 2, 16
_NW = _NC * _NS
_CHUNK = 384
_NBUF = 2

_TC_BLK = 512
_TC_ROWS = 192512


def _sc_gather(table, idx_flat, n_idx, dim, row0_out):
    m = idx_flat.shape[1]
    per_w = m // _NW
    assert m % _NW == 0 and per_w % 128 == 0
    n_full = per_w // _CHUNK
    tail = per_w - n_full * _CHUNK
    n_chunk = n_full + (1 if tail else 0)
    assert tail % 128 == 0 and n_chunk >= 2 * _NBUF

    def size_of(k):
        return _CHUNK if k < n_full else tail

    mesh = plsc.VectorSubcoreMesh(core_axis_name="core", subcore_axis_name="subcore")

    @pl.kernel(
        out_type=jax.ShapeDtypeStruct((m, dim), table.dtype),
        mesh=mesh,
        scratch_types=[
            pltpu.VMEM((per_w,), jnp.int32),
            pltpu.VMEM((_NBUF, _CHUNK, dim), table.dtype),
            pltpu.SemaphoreType.DMA,
            pltpu.SemaphoreType.DMA((_NBUF,)),
            pltpu.SemaphoreType.DMA((_NBUF,)),
        ],
    )
    def gather_kernel(table_hbm, idx_hbm, out_hbm, idx_v, bufs, sidx, sg, ss):
        wid = lax.axis_index("subcore") * _NC + lax.axis_index("core")
        base = pl.multiple_of(wid * per_w, 128)

        pltpu.async_copy(idx_hbm.at[0, pl.ds(base, per_w)], idx_v, sidx).wait()

        def gather_copy(k, b, sz=_CHUNK):
            off = pl.multiple_of(k * _CHUNK, 128)
            return pltpu.make_async_copy(
                table_hbm.at[idx_v.at[pl.ds(off, sz)]],
                bufs.at[b, pl.ds(0, sz)],
                sg.at[b],
            )

        def store_copy(k, b, sz=_CHUNK):
            r0 = pl.multiple_of(base + k * _CHUNK, 128)
            return pltpu.make_async_copy(
                bufs.at[b, pl.ds(0, sz)],
                out_hbm.at[pl.ds(r0, sz)],
                ss.at[b],
            )

        def ring_body(k, j, ksz=_CHUNK, psz=_CHUNK, wsz=_CHUNK):
            b2 = (j + 1) % _NBUF
            gather_copy(k - _NBUF + 1, b2, psz).wait()
            store_copy(k - _NBUF + 1, b2, psz).start()
            store_copy(k - _NBUF, j, wsz).wait()
            gather_copy(k, j, ksz).start()

        for j in range(_NBUF):
            gather_copy(j, j).start()
        gather_copy(0, 0).wait()
        store_copy(0, 0).start()

        steady_end = _NBUF + ((n_full - _NBUF) // _NBUF) * _NBUF

        @pl.loop(_NBUF, steady_end, step=_NBUF)
        def _(k0):
            for j in range(_NBUF):
                ring_body(k0 + j, j)

        for k in range(steady_end, n_chunk):
            ring_body(k, k % _NBUF, ksz=size_of(k), psz=size_of(k - _NBUF + 1),
                      wsz=size_of(k - _NBUF))

        for k in range(n_chunk - _NBUF + 1, n_chunk):
            gather_copy(k, k % _NBUF, size_of(k)).wait()
            store_copy(k, k % _NBUF, size_of(k)).start()
        for k in range(n_chunk - _NBUF, n_chunk):
            store_copy(k, k % _NBUF, size_of(k)).wait()

    del row0_out
    return gather_kernel(table, idx_flat)


def _tc_gather(table, idx_flat, vocab, dim):
    t = idx_flat.shape[1]
    assert t % _TC_BLK == 0

    def body(idx_sref, table_ref, out_ref):
        for i0 in range(0, _TC_BLK, 8):
            rows = [
                table_ref[pl.ds(idx_sref[0, i0 + i], 1), :] for i in range(8)
            ]
            out_ref[pl.ds(i0, 8), :] = jnp.concatenate(rows, axis=0)

    return pl.pallas_call(
        body,
        grid=(t // _TC_BLK,),
        in_specs=[
            pl.BlockSpec((1, _TC_BLK), lambda i: (0, i), memory_space=pltpu.SMEM),
            pl.BlockSpec((vocab, dim), lambda i: (0, 0)),
        ],
        out_specs=pl.BlockSpec((_TC_BLK, dim), lambda i: (i, 0)),
        out_shape=jax.ShapeDtypeStruct((t, dim), table.dtype),
    )(idx_flat, table)


def kernel(x, embedding_weight):
    batch, seq = x.shape
    vocab, dim = embedding_weight.shape
    n_idx = batch * seq
    idx_flat = x.reshape(1, n_idx).astype(jnp.int32)
    sc_out = _sc_gather(
        embedding_weight, idx_flat[:, _TC_ROWS:], n_idx, dim, _TC_ROWS
    )
    tc_out = _tc_gather(embedding_weight, idx_flat[:, :_TC_ROWS], vocab, dim)
    out = jnp.concatenate([tc_out, sc_out], axis=0)
    return out.reshape(batch, seq, dim)


# speedup vs baseline: 1.4230x; 1.4230x over previous
import jax
import jax.numpy as jnp
from jax import lax
from jax.experimental.compute_on import compute_on
from jax.experimental import pallas as pl
from jax.experimental.pallas import tpu as pltpu
from jax.experimental.pallas import tpu_sc as plsc

_NC, _NS = 2, 16
_NW = _NC * _NS
_CHUNK = 384
_NBUF = 2

_TC_BLK = 512
_TC_ROWS = 192512


def _sc_gather(table, idx_flat, n_idx, dim, row0_out):
    m = idx_flat.shape[1]
    assert row0_out % 128 == 0
    per_w = m // _NW
    assert m % _NW == 0 and per_w % 128 == 0
    n_full = per_w // _CHUNK
    tail = per_w - n_full * _CHUNK
    n_chunk = n_full + (1 if tail else 0)
    assert tail % 128 == 0 and n_chunk >= 2 * _NBUF

    def size_of(k):
        return _CHUNK if k < n_full else tail

    mesh = plsc.VectorSubcoreMesh(core_axis_name="core", subcore_axis_name="subcore")

    @pl.kernel(
        out_type=jax.ShapeDtypeStruct((n_idx, dim), table.dtype),
        mesh=mesh,
        scratch_types=[
            pltpu.VMEM((per_w,), jnp.int32),
            pltpu.VMEM((_NBUF, _CHUNK, dim), table.dtype),
            pltpu.SemaphoreType.DMA,
            pltpu.SemaphoreType.DMA((_NBUF,)),
            pltpu.SemaphoreType.DMA((_NBUF,)),
        ],
    )
    def gather_kernel(table_hbm, idx_hbm, out_hbm, idx_v, bufs, sidx, sg, ss):
        wid = lax.axis_index("subcore") * _NC + lax.axis_index("core")
        base = pl.multiple_of(wid * per_w, 128)

        pltpu.async_copy(idx_hbm.at[0, pl.ds(base, per_w)], idx_v, sidx).wait()

        def gather_copy(k, b, sz=_CHUNK):
            off = pl.multiple_of(k * _CHUNK, 128)
            return pltpu.make_async_copy(
                table_hbm.at[idx_v.at[pl.ds(off, sz)]],
                bufs.at[b, pl.ds(0, sz)],
                sg.at[b],
            )

        def store_copy(k, b, sz=_CHUNK):
            r0 = pl.multiple_of(row0_out + base + k * _CHUNK, 128)
            return pltpu.make_async_copy(
                bufs.at[b, pl.ds(0, sz)],
                out_hbm.at[pl.ds(r0, sz)],
                ss.at[b],
            )

        def ring_body(k, j, ksz=_CHUNK, psz=_CHUNK, wsz=_CHUNK):
            b2 = (j + 1) % _NBUF
            gather_copy(k - _NBUF + 1, b2, psz).wait()
            store_copy(k - _NBUF + 1, b2, psz).start()
            store_copy(k - _NBUF, j, wsz).wait()
            gather_copy(k, j, ksz).start()

        for j in range(_NBUF):
            gather_copy(j, j).start()
        gather_copy(0, 0).wait()
        store_copy(0, 0).start()

        steady_end = _NBUF + ((n_full - _NBUF) // _NBUF) * _NBUF

        @pl.loop(_NBUF, steady_end, step=_NBUF)
        def _(k0):
            for j in range(_NBUF):
                ring_body(k0 + j, j)

        for k in range(steady_end, n_chunk):
            ring_body(k, k % _NBUF, ksz=size_of(k), psz=size_of(k - _NBUF + 1),
                      wsz=size_of(k - _NBUF))

        for k in range(n_chunk - _NBUF + 1, n_chunk):
            gather_copy(k, k % _NBUF, size_of(k)).wait()
            store_copy(k, k % _NBUF, size_of(k)).start()
        for k in range(n_chunk - _NBUF, n_chunk):
            store_copy(k, k % _NBUF, size_of(k)).wait()

    return gather_kernel(table, idx_flat)


def _tc_gather(table, idx_flat, vocab, dim):
    t = idx_flat.shape[1]
    assert t % _TC_BLK == 0

    def body(idx_sref, table_ref, out_ref):
        for i0 in range(0, _TC_BLK, 8):
            rows = [
                table_ref[pl.ds(idx_sref[0, i0 + i], 1), :] for i in range(8)
            ]
            out_ref[pl.ds(i0, 8), :] = jnp.concatenate(rows, axis=0)

    return pl.pallas_call(
        body,
        grid=(t // _TC_BLK,),
        in_specs=[
            pl.BlockSpec((1, _TC_BLK), lambda i: (0, i), memory_space=pltpu.SMEM),
            pl.BlockSpec((vocab, dim), lambda i: (0, 0)),
        ],
        out_specs=pl.BlockSpec((_TC_BLK, dim), lambda i: (i, 0)),
        out_shape=jax.ShapeDtypeStruct((t, dim), table.dtype),
    )(idx_flat, table)


def kernel(x, embedding_weight):
    batch, seq = x.shape
    vocab, dim = embedding_weight.shape
    n_idx = batch * seq
    idx_flat = x.reshape(1, n_idx).astype(jnp.int32)
    with compute_on("tpu_sparsecore"):
        sc_full = _sc_gather(
            embedding_weight, idx_flat[:, _TC_ROWS:], n_idx, dim, _TC_ROWS
        )
    tc_out = _tc_gather(embedding_weight, idx_flat[:, :_TC_ROWS], vocab, dim)
    out = lax.dynamic_update_slice(sc_full, tc_out, (0, 0))
    return out.reshape(batch, seq, dim)


# speedup vs baseline: 2.0167x; 1.4172x over previous
import jax
import jax.numpy as jnp
from jax.experimental import pallas as pl
from jax.experimental.pallas import tpu as pltpu
from jax.experimental.pallas import tpu_sc as plsc

_WINDOW = 256


def _gather_rows(table, idx_flat, n_idx, dim):
    mesh = plsc.VectorSubcoreMesh(core_axis_name="core", subcore_axis_name="subcore")

    @pl.kernel(
        out_type=jax.ShapeDtypeStruct((n_idx, dim), table.dtype),
        mesh=mesh,
    )
    def gather_kernel(table_hbm, idx_hbm, out_hbm):
        def body(idx_vmem, out_vmem):
            pltpu.sync_copy(table_hbm.at[idx_vmem.at[0]], out_vmem)

        pltpu.emit_pipeline(
            body,
            grid=(n_idx // _WINDOW,),
            in_specs=[pl.BlockSpec((1, _WINDOW), index_map=lambda i: (0, i))],
            out_specs=[pl.BlockSpec((_WINDOW, dim), index_map=lambda i: (i, 0))],
            core_axis_name=("core", "subcore"),
            dimension_semantics=(pltpu.PARALLEL,),
        )(idx_hbm, out_hbm)

    return gather_kernel(table, idx_flat)


def kernel(x, embedding_weight):
    batch, seq = x.shape
    vocab, dim = embedding_weight.shape
    n_idx = batch * seq
    idx_flat = x.reshape(1, n_idx).astype(jnp.int32)
    out = _gather_rows(embedding_weight, idx_flat, n_idx, dim)
    return out.reshape(batch, seq, dim)
